# baseline (speedup 1.0000x reference)
import dataclasses
import functools

import jax
import jax.numpy as jnp
from jax import lax
from jax.experimental import pallas as pl
from jax.experimental.pallas import tpu as pltpu
from jax.experimental.pallas import tpu_sc as plsc

_NEG_SLOPE = 0.01
_NW = 32
_NT = 16
_CH = 128


def _leaky(v):
    return jnp.where(v >= 0, v, _NEG_SLOPE * v)


def _stage_a(x, W1, b1):
    N, D = x.shape
    F = W1.shape[1]
    BN = 1000

    def body(x_ref, w_ref, b_ref, a_ref, m_ref):
        xv = x_ref[...]
        a_ref[...] = jnp.dot(xv, w_ref[0:D, :],
                             preferred_element_type=jnp.float32,
                             precision=lax.Precision.HIGHEST) + b_ref[...]
        m_ref[...] = jnp.dot(xv, w_ref[D:, :],
                             preferred_element_type=jnp.float32,
                             precision=lax.Precision.HIGHEST)

    return pl.pallas_call(
        body,
        grid=(N // BN,),
        in_specs=[pl.BlockSpec((BN, D), lambda i: (i, 0)),
                  pl.BlockSpec((2 * D, F), lambda i: (0, 0)),
                  pl.BlockSpec((1, F), lambda i: (0, 0))],
        out_specs=[pl.BlockSpec((BN, F), lambda i: (i, 0)),
                   pl.BlockSpec((BN, F), lambda i: (i, 0))],
        out_shape=[jax.ShapeDtypeStruct((N, F), jnp.float32),
                   jax.ShapeDtypeStruct((N, F), jnp.float32)],
    )(x, W1, b1.reshape(1, F))


def _stage_b(a1, p):
    N, F = a1.shape
    BN = 1000

    def body(a_ref, p_ref, o_ref):
        v = a_ref[...] + p_ref[0] + p_ref[1]
        o_ref[...] = _leaky(v)

    return pl.pallas_call(
        body,
        grid=(N // BN,),
        in_specs=[pl.BlockSpec((BN, F), lambda i: (i, 0)),
                  pl.BlockSpec((2, BN, F), lambda i: (0, i, 0))],
        out_specs=pl.BlockSpec((BN, F), lambda i: (i, 0)),
        out_shape=jax.ShapeDtypeStruct((N, F), jnp.float32),
    )(a1, p)


def _stage_c(h1, p, W2, b2):
    N, F = h1.shape
    OUTD = W2.shape[1]
    BN = 1000

    def body(h_ref, p_ref, w_ref, b_ref, o_ref):
        s = p_ref[0] + p_ref[1]
        v = (jnp.dot(h_ref[...], w_ref[0:F, :],
                     preferred_element_type=jnp.float32,
                     precision=lax.Precision.HIGHEST)
             + jnp.dot(s, w_ref[F:, :],
                       preferred_element_type=jnp.float32,
                       precision=lax.Precision.HIGHEST)
             + b_ref[...])
        o_ref[...] = _leaky(v)

    return pl.pallas_call(
        body,
        grid=(N // BN,),
        in_specs=[pl.BlockSpec((BN, F), lambda i: (i, 0)),
                  pl.BlockSpec((2, BN, F), lambda i: (0, i, 0)),
                  pl.BlockSpec((2 * F, OUTD), lambda i: (0, 0)),
                  pl.BlockSpec((1, OUTD), lambda i: (0, 0))],
        out_specs=pl.BlockSpec((BN, OUTD), lambda i: (i, 0)),
        out_shape=jax.ShapeDtypeStruct((N, OUTD), jnp.float32),
    )(h1, p, W2, b2.reshape(1, OUTD))


def _make_spmv(N_pad, F, NCH0, NCH1):
    mesh = plsc.VectorSubcoreMesh(core_axis_name="c", subcore_axis_name="s",
                                  num_cores=2, num_subcores=_NT)
    cp = pltpu.CompilerParams()
    if "needs_layout_passes" in pltpu.CompilerParams.__dataclass_fields__:
        cp = dataclasses.replace(cp, needs_layout_passes=False)
    rows_per_tile = N_pad // _NT
    nz = rows_per_tile // _CH

    @functools.partial(
        pl.kernel,
        out_type=jax.ShapeDtypeStruct((2, N_pad, F), jnp.float32),
        mesh=mesh,
        scratch_types=[
            pltpu.VMEM((NCH0, _CH), jnp.int32),
            pltpu.VMEM((2 * _CH,), jnp.int32),
            pltpu.VMEM((2 * _CH,), jnp.float32),
            pltpu.VMEM((_CH, F), jnp.float32),
            pltpu.VMEM((_CH, F), jnp.float32),
            pltpu.VMEM_SHARED((N_pad, F), jnp.float32),
            pltpu.SemaphoreType.DMA,
            pltpu.SemaphoreType.DMA,
            pltpu.SemaphoreType.DMA,
            pltpu.SemaphoreType.DMA,
            pltpu.SemaphoreType.DMA,
            pltpu.SemaphoreType.DMA,
        ],
        compiler_params=cp,
    )
    def spmv(m_hbm, col_hbm, row_hbm, w_hbm, out_hbm,
             row_v, col_p, w_p, msg0_v, msg1_v, acc_sh,
             gsem0, gsem1, ssem0, ssem1, icsem, iwsem):
        cid = lax.axis_index("c")
        sid = lax.axis_index("s")
        wid = cid * _NT + sid
        ncw = jnp.where(cid == 0, NCH0, NCH1)

        pltpu.sync_copy(row_hbm.at[pl.ds(wid * NCH0, NCH0)], row_v)

        zero = jnp.zeros((16,), jnp.float32)

        @pl.loop(0, _CH)
        def _(i):
            for f in range(F // 16):
                msg0_v[i, pl.ds(f * 16, 16)] = zero

        @pl.loop(0, nz)
        def _(j):
            pltpu.sync_copy(
                msg0_v,
                acc_sh.at[pl.ds(sid * rows_per_tile + j * _CH, _CH)])

        plsc.subcore_barrier()

        def scale(eoff, buf):
            @pl.loop(0, _CH, unroll=4)
            def _(e):
                ee = jnp.full((16,), eoff + e, jnp.int32)
                wv = plsc.load_gather(w_p, [ee])
                for f in range(F // 16):
                    buf[e, pl.ds(f * 16, 16)] = buf[e, pl.ds(f * 16, 16)] * wv

        base0 = wid * NCH0 * _CH
        pltpu.sync_copy(col_hbm.at[pl.ds(base0, 2 * _CH)], col_p)
        pltpu.sync_copy(w_hbm.at[pl.ds(base0, 2 * _CH)], w_p)

        def wait_scatter(buf, sem):
            pltpu.make_async_copy(buf, acc_sh.at[row_v.at[0]], sem).wait()

        @pl.loop(0, NCH0 // 2)
        def _(i):
            j0 = 2 * i
            j1 = j0 + 1

            @pl.when(j0 < ncw)
            def _():
                @pl.when(i > 0)
                def _():
                    pltpu.make_async_copy(
                        col_hbm.at[pl.ds(base0, 2 * _CH)], col_p,
                        icsem).wait()
                    wait_scatter(msg0_v, ssem0)

                g0 = pltpu.async_copy(m_hbm.at[col_p.at[pl.ds(0, _CH)]],
                                      msg0_v, gsem0)

                @pl.when(i > 0)
                def _():
                    wait_scatter(msg1_v, ssem1)

                g1 = pltpu.async_copy(m_hbm.at[col_p.at[pl.ds(_CH, _CH)]],
                                      msg1_v, gsem1)
                g0.wait()

                @pl.when(i > 0)
                def _():
                    pltpu.make_async_copy(
                        w_hbm.at[pl.ds(base0, 2 * _CH)], w_p, iwsem).wait()

                scale(0, msg0_v)
                pltpu.async_copy(msg0_v, acc_sh.at[row_v.at[j0]], ssem0,
                                 add=True)
                g1.wait()

                nbase = (wid * NCH0 + j0 + 2) * _CH

                @pl.when(j0 + 2 < ncw)
                def _():
                    pltpu.async_copy(col_hbm.at[pl.ds(nbase, 2 * _CH)],
                                     col_p, icsem)

                scale(_CH, msg1_v)
                pltpu.async_copy(msg1_v, acc_sh.at[row_v.at[j1]], ssem1,
                                 add=True)

                @pl.when(j0 + 2 < ncw)
                def _():
                    pltpu.async_copy(w_hbm.at[pl.ds(nbase, 2 * _CH)],
                                     w_p, iwsem)

        @pl.when(ncw > 0)
        def _():
            wait_scatter(msg0_v, ssem0)
            wait_scatter(msg1_v, ssem1)

        plsc.subcore_barrier()

        base = sid * rows_per_tile
        pltpu.sync_copy(acc_sh.at[pl.ds(base, rows_per_tile)],
                        out_hbm.at[cid, pl.ds(base, rows_per_tile)])

    return spmv


def kernel(x, edge_index, edge_weight, W1, b1, W2, b2):
    N, D = x.shape
    F = W1.shape[1]
    E = edge_weight.shape[0]

    NCHT = 2 * (-(-E // (_NW * _CH)))
    NCH0 = min(NCHT, ((NCHT + 7) // 8) * 8)
    NCH1 = NCHT - NCH0
    E0 = _NT * NCH0 * _CH
    E1_pad = _NT * NCH1 * _CH
    assert E <= E0 + E1_pad

    def pack(a):
        a = jnp.pad(a, (0, E0 + E1_pad - E))
        a0 = a[:E0].reshape(_NT, NCH0, _CH)
        a1 = a[E0:].reshape(_NT, NCH1, _CH)
        a1 = jnp.pad(a1, ((0, 0), (0, NCH0 - NCH1), (0, 0)))
        return jnp.concatenate([a0, a1], axis=0).reshape(_NW * NCH0, _CH)

    row = pack(edge_index[0])
    col = pack(edge_index[1]).reshape(-1)
    w = pack(edge_weight).reshape(-1)

    N_pad = -(-N // (_NT * _CH)) * (_NT * _CH)
    spmv = _make_spmv(N_pad, F, NCH0, NCH1)

    a1, m1 = _stage_a(x, W1, b1)
    p1 = spmv(m1, col, row, w)
    h1 = _stage_b(a1, p1)
    p2 = spmv(h1, col, row, w)
    return _stage_c(h1, p2, W2, b2)


# speedup vs baseline: 1.7844x; 1.7844x over previous
import dataclasses
import functools

import jax
import jax.numpy as jnp
from jax import lax
from jax.experimental import pallas as pl
from jax.experimental.pallas import tpu as pltpu
from jax.experimental.pallas import tpu_sc as plsc

_NEG_SLOPE = 0.01
_NW = 32
_NT = 16
_CH = 128


def _leaky(v):
    return jnp.where(v >= 0, v, _NEG_SLOPE * v)


def _stage_a(x, W1, b1):
    N, D = x.shape
    F = W1.shape[1]
    BN = 1000

    def body(x_ref, w_ref, b_ref, a_ref, m_ref):
        xv = x_ref[...]
        a_ref[...] = jnp.dot(xv, w_ref[0:D, :],
                             preferred_element_type=jnp.float32,
                             precision=lax.Precision.HIGHEST) + b_ref[...]
        m_ref[...] = jnp.dot(xv, w_ref[D:, :],
                             preferred_element_type=jnp.float32,
                             precision=lax.Precision.HIGHEST)

    return pl.pallas_call(
        body,
        grid=(N // BN,),
        in_specs=[pl.BlockSpec((BN, D), lambda i: (i, 0)),
                  pl.BlockSpec((2 * D, F), lambda i: (0, 0)),
                  pl.BlockSpec((1, F), lambda i: (0, 0))],
        out_specs=[pl.BlockSpec((BN, F), lambda i: (i, 0)),
                   pl.BlockSpec((BN, F), lambda i: (i, 0))],
        out_shape=[jax.ShapeDtypeStruct((N, F), jnp.float32),
                   jax.ShapeDtypeStruct((N, F), jnp.float32)],
    )(x, W1, b1.reshape(1, F))


def _stage_b(a1, p):
    N, F = a1.shape
    BN = 1000

    def body(a_ref, p_ref, o_ref):
        v = a_ref[...] + p_ref[0] + p_ref[1]
        o_ref[...] = _leaky(v)

    return pl.pallas_call(
        body,
        grid=(N // BN,),
        in_specs=[pl.BlockSpec((BN, F), lambda i: (i, 0)),
                  pl.BlockSpec((2, BN, F), lambda i: (0, i, 0))],
        out_specs=pl.BlockSpec((BN, F), lambda i: (i, 0)),
        out_shape=jax.ShapeDtypeStruct((N, F), jnp.float32),
    )(a1, p)


def _stage_c(h1, p, W2, b2):
    N, F = h1.shape
    OUTD = W2.shape[1]
    BN = 1000

    def body(h_ref, p_ref, w_ref, b_ref, o_ref):
        s = p_ref[0] + p_ref[1]
        v = (jnp.dot(h_ref[...], w_ref[0:F, :],
                     preferred_element_type=jnp.float32,
                     precision=lax.Precision.HIGHEST)
             + jnp.dot(s, w_ref[F:, :],
                       preferred_element_type=jnp.float32,
                       precision=lax.Precision.HIGHEST)
             + b_ref[...])
        o_ref[...] = _leaky(v)

    return pl.pallas_call(
        body,
        grid=(N // BN,),
        in_specs=[pl.BlockSpec((BN, F), lambda i: (i, 0)),
                  pl.BlockSpec((2, BN, F), lambda i: (0, i, 0)),
                  pl.BlockSpec((2 * F, OUTD), lambda i: (0, 0)),
                  pl.BlockSpec((1, OUTD), lambda i: (0, 0))],
        out_specs=pl.BlockSpec((BN, OUTD), lambda i: (i, 0)),
        out_shape=jax.ShapeDtypeStruct((N, OUTD), jnp.float32),
    )(h1, p, W2, b2.reshape(1, OUTD))


def _make_spmv(N_pad, F, NCH0, NCH1):
    mesh = plsc.VectorSubcoreMesh(core_axis_name="c", subcore_axis_name="s",
                                  num_cores=2, num_subcores=_NT)
    cp = pltpu.CompilerParams()
    if "needs_layout_passes" in pltpu.CompilerParams.__dataclass_fields__:
        cp = dataclasses.replace(cp, needs_layout_passes=False)
    rows_per_tile = N_pad // _NT
    nz = rows_per_tile // _CH

    @functools.partial(
        pl.kernel,
        out_type=jax.ShapeDtypeStruct((2, N_pad, F), jnp.float32),
        mesh=mesh,
        scratch_types=[
            pltpu.VMEM((NCH0, _CH), jnp.int32),
            pltpu.VMEM((2 * _CH,), jnp.int32),
            pltpu.VMEM((2 * _CH,), jnp.float32),
            pltpu.VMEM((_CH, F), jnp.float32),
            pltpu.VMEM((_CH, F), jnp.float32),
            pltpu.VMEM_SHARED((N_pad, F), jnp.float32),
            pltpu.SemaphoreType.DMA,
            pltpu.SemaphoreType.DMA,
            pltpu.SemaphoreType.DMA,
            pltpu.SemaphoreType.DMA,
            pltpu.SemaphoreType.DMA,
            pltpu.SemaphoreType.DMA,
        ],
        compiler_params=cp,
    )
    def spmv(m_hbm, col_hbm, row_hbm, w_hbm, out_hbm,
             row_v, col_p, w_p, msg0_v, msg1_v, acc_sh,
             gsem0, gsem1, ssem0, ssem1, icsem, iwsem):
        cid = lax.axis_index("c")
        sid = lax.axis_index("s")
        wid = cid * _NT + sid
        ncw = jnp.where(cid == 0, NCH0, NCH1)

        pltpu.sync_copy(row_hbm.at[pl.ds(wid * NCH0, NCH0)], row_v)

        zero = jnp.zeros((16,), jnp.float32)

        @pl.loop(0, _CH)
        def _(i):
            for f in range(F // 16):
                msg0_v[i, pl.ds(f * 16, 16)] = zero

        @pl.loop(0, nz)
        def _(j):
            pltpu.sync_copy(
                msg0_v,
                acc_sh.at[pl.ds(sid * rows_per_tile + j * _CH, _CH)])

        plsc.subcore_barrier()

        def scale(eoff, buf):
            @pl.loop(0, _CH, unroll=4)
            def _(e):
                ee = jnp.full((16,), eoff + e, jnp.int32)
                wv = plsc.load_gather(w_p, [ee])
                for f in range(F // 16):
                    buf[e, pl.ds(f * 16, 16)] = buf[e, pl.ds(f * 16, 16)] * wv

        base0 = wid * NCH0 * _CH
        pltpu.sync_copy(col_hbm.at[pl.ds(base0, 2 * _CH)], col_p)
        pltpu.sync_copy(w_hbm.at[pl.ds(base0, 2 * _CH)], w_p)

        def wait_scatter(buf, sem):
            pltpu.make_async_copy(buf, acc_sh.at[row_v.at[0]], sem).wait()

        @pl.loop(0, NCH0 // 2)
        def _(i):
            j0 = 2 * i
            j1 = j0 + 1

            @pl.when(j0 < ncw)
            def _():
                @pl.when(i > 0)
                def _():
                    pltpu.make_async_copy(
                        col_hbm.at[pl.ds(base0, 2 * _CH)], col_p,
                        icsem).wait()
                    wait_scatter(msg0_v, ssem0)

                g0 = pltpu.async_copy(m_hbm.at[col_p.at[pl.ds(0, _CH)]],
                                      msg0_v, gsem0)

                @pl.when(i > 0)
                def _():
                    wait_scatter(msg1_v, ssem1)

                g1 = pltpu.async_copy(m_hbm.at[col_p.at[pl.ds(_CH, _CH)]],
                                      msg1_v, gsem1)
                g0.wait()

                @pl.when(i > 0)
                def _():
                    pltpu.make_async_copy(
                        w_hbm.at[pl.ds(base0, 2 * _CH)], w_p, iwsem).wait()

                scale(0, msg0_v)
                pltpu.async_copy(msg0_v, acc_sh.at[row_v.at[j0]], ssem0,
                                 add=True)
                g1.wait()

                nbase = (wid * NCH0 + j0 + 2) * _CH

                @pl.when(j0 + 2 < ncw)
                def _():
                    pltpu.async_copy(col_hbm.at[pl.ds(nbase, 2 * _CH)],
                                     col_p, icsem)

                scale(_CH, msg1_v)
                pltpu.async_copy(msg1_v, acc_sh.at[row_v.at[j1]], ssem1,
                                 add=True)

                @pl.when(j0 + 2 < ncw)
                def _():
                    pltpu.async_copy(w_hbm.at[pl.ds(nbase, 2 * _CH)],
                                     w_p, iwsem)

        @pl.when(ncw > 0)
        def _():
            wait_scatter(msg0_v, ssem0)
            wait_scatter(msg1_v, ssem1)

        plsc.subcore_barrier()

        base = sid * rows_per_tile
        pltpu.sync_copy(acc_sh.at[pl.ds(base, rows_per_tile)],
                        out_hbm.at[cid, pl.ds(base, rows_per_tile)])

    return spmv


def kernel(x, edge_index, edge_weight, W1, b1, W2, b2):
    N, D = x.shape
    F = W1.shape[1]
    E = edge_weight.shape[0]

    NCHT = 2 * (-(-E // (_NW * _CH)))
    NCH0 = max(8, min(NCHT - 8, ((NCHT * 7 // 8 + 4) // 8) * 8))
    NCH1 = NCHT - NCH0
    E0 = _NT * NCH0 * _CH
    E1_pad = _NT * NCH1 * _CH
    assert E <= E0 + E1_pad

    def pack(a):
        a = jnp.pad(a, (0, E0 + E1_pad - E))
        a0 = a[:E0].reshape(_NT, NCH0, _CH)
        a1 = a[E0:].reshape(_NT, NCH1, _CH)
        a1 = jnp.pad(a1, ((0, 0), (0, NCH0 - NCH1), (0, 0)))
        return jnp.concatenate([a0, a1], axis=0).reshape(_NW * NCH0, _CH)

    row = pack(edge_index[0])
    col = pack(edge_index[1]).reshape(-1)
    w = pack(edge_weight).reshape(-1)

    N_pad = -(-N // (_NT * _CH)) * (_NT * _CH)
    spmv = _make_spmv(N_pad, F, NCH0, NCH1)

    a1, m1 = _stage_a(x, W1, b1)
    p1 = spmv(m1, col, row, w)
    h1 = _stage_b(a1, p1)
    p2 = spmv(h1, col, row, w)
    return _stage_c(h1, p2, W2, b2)
